# baseline (speedup 1.0000x reference)
import jax
import jax.numpy as jnp
from jax import lax
from jax.experimental import pallas as pl
from jax.experimental.pallas import tpu as pltpu
from jax.experimental.pallas import tpu_sc as plsc

N = 10000
E = 160000
D = 128
NC, NS = 2, 16
NW = NC * NS
EPW = E // NW
CHUNK = 125
NCHUNK = EPW // CHUNK
CCH = 40
NCCH = EPW // CCH
NPAD = 10240
RPT = NPAD // NS
DEGW = 128

_SC_MESH = plsc.VectorSubcoreMesh(
    core_axis_name="c", subcore_axis_name="s", num_cores=NC, num_subcores=NS)


def _seg_sum_body(x_hbm, srcs_hbm, dsts_hbm, zeros_hbm,
                  agg_hbm, idxs, idxd, rows0, rows1, acc, sem0, sem1,
                  semS0, semS1):
  cid = lax.axis_index("c")
  sid = lax.axis_index("s")
  wid = cid * NS + sid
  pltpu.sync_copy(zeros_hbm.at[pl.ds(sid * RPT, RPT)],
                  acc.at[pl.ds(sid * RPT, RPT)])
  pltpu.sync_copy(srcs_hbm.at[wid], idxs)
  pltpu.sync_copy(dsts_hbm.at[wid], idxd)
  plsc.subcore_barrier()

  pltpu.async_copy(x_hbm.at[idxs.at[0]], rows0, sem0)
  pltpu.async_copy(x_hbm.at[idxs.at[1]], rows1, sem1)

  def body(p, carry):
    j0 = 2 * p
    pltpu.make_async_copy(x_hbm.at[idxs.at[j0]], rows0, sem0).wait()
    pltpu.async_copy(rows0, acc.at[idxd.at[j0]], semS0, add=True)

    pltpu.make_async_copy(x_hbm.at[idxs.at[j0 + 1]], rows1, sem1).wait()
    pltpu.async_copy(rows1, acc.at[idxd.at[j0 + 1]], semS1, add=True)

    pltpu.make_async_copy(rows0, acc.at[idxd.at[j0]], semS0).wait()

    @pl.when(j0 + 2 < NCHUNK)
    def _():
      pltpu.async_copy(x_hbm.at[idxs.at[j0 + 2]], rows0, sem0)

    pltpu.make_async_copy(rows1, acc.at[idxd.at[j0 + 1]], semS1).wait()

    @pl.when(j0 + 3 < NCHUNK)
    def _():
      pltpu.async_copy(x_hbm.at[idxs.at[j0 + 3]], rows1, sem1)

    return carry

  lax.fori_loop(0, NCHUNK // 2, body, 0)
  plsc.subcore_barrier()
  pltpu.sync_copy(acc.at[pl.ds(sid * RPT, RPT)],
                  agg_hbm.at[cid, pl.ds(sid * RPT, RPT)])


_seg_sum = pl.kernel(
    _seg_sum_body,
    out_type=jax.ShapeDtypeStruct((NC, NPAD, D), jnp.float32),
    mesh=_SC_MESH,
    scratch_types=[
        pltpu.VMEM((NCHUNK, CHUNK), jnp.int32),
        pltpu.VMEM((NCHUNK, CHUNK), jnp.int32),
        pltpu.VMEM((CHUNK, D), jnp.float32),
        pltpu.VMEM((CHUNK, D), jnp.float32),
        pltpu.VMEM_SHARED((NPAD, D), jnp.float32),
        pltpu.SemaphoreType.DMA,
        pltpu.SemaphoreType.DMA,
        pltpu.SemaphoreType.DMA,
        pltpu.SemaphoreType.DMA,
    ],
)


def _degree_body(dsts_hbm, ones_hbm, zerosd_hbm, deg_hbm,
                 idxd, ones_v, dacc, sem):
  cid = lax.axis_index("c")
  sid = lax.axis_index("s")
  wid = cid * NS + sid
  pltpu.sync_copy(zerosd_hbm.at[pl.ds(sid * RPT, RPT)],
                  dacc.at[pl.ds(sid * RPT, RPT)])
  pltpu.sync_copy(dsts_hbm.at[wid], idxd)
  pltpu.sync_copy(ones_hbm, ones_v)
  plsc.subcore_barrier()

  def body(j, carry):
    pltpu.async_copy(ones_v, dacc.at[idxd.at[j]], sem, add=True)
    return carry

  lax.fori_loop(0, NCHUNK, body, 0)

  def drain(j, carry):
    pltpu.make_async_copy(ones_v, dacc.at[idxd.at[j]], sem).wait()
    return carry

  lax.fori_loop(0, NCHUNK, drain, 0)
  plsc.subcore_barrier()
  pltpu.sync_copy(dacc.at[pl.ds(sid * RPT, RPT)],
                  deg_hbm.at[cid, pl.ds(sid * RPT, RPT)])


_degree = pl.kernel(
    _degree_body,
    out_type=jax.ShapeDtypeStruct((NC, NPAD, DEGW), jnp.float32),
    mesh=_SC_MESH,
    scratch_types=[
        pltpu.VMEM((NCHUNK, CHUNK), jnp.int32),
        pltpu.VMEM((CHUNK, DEGW), jnp.float32),
        pltpu.VMEM_SHARED((NPAD, DEGW), jnp.float32),
        pltpu.SemaphoreType.DMA,
    ],
)


GRP = 5
NG = NCCH // GRP
GR = GRP * CCH
NG0 = 12
NG1 = NG - NG0


def _make_edge_combine(ng):
  nch = ng * GRP

  def body_fn(a_hbm, b_hbm, srcs_hbm, dsts_hbm, c_hbm,
              idxs, idxd, rows0, rows1, semA0, semA1, semB, semC0, semC1):
    cid = lax.axis_index("c")
    sid = lax.axis_index("s")
    wid = cid * NS + sid
    pltpu.sync_copy(srcs_hbm.at[wid], idxs)
    pltpu.sync_copy(dsts_hbm.at[wid], idxd)
    base = wid * (ng * GR)

    def issue_a(lg, buf, sem):
      for k in range(GRP):
        pltpu.async_copy(a_hbm.at[idxs.at[lg * GRP + k]],
                         buf.at[pl.ds(k * CCH, CCH)], sem)

    def process(lg, buf, sem, semc):
      for k in range(GRP):
        pltpu.make_async_copy(a_hbm.at[idxs.at[lg * GRP + k]],
                              buf.at[pl.ds(k * CCH, CCH)], sem).wait()
      descs = [
          pltpu.async_copy(b_hbm.at[idxd.at[lg * GRP + k]],
                           buf.at[pl.ds(k * CCH, CCH)], semB, add=True)
          for k in range(GRP)
      ]
      for desc in descs:
        desc.wait()
      pltpu.async_copy(buf, c_hbm.at[pl.ds(base + lg * GR, GR)], semc)

    def wait_c(lg, buf, semc):
      pltpu.make_async_copy(buf, c_hbm.at[pl.ds(base + lg * GR, GR)],
                            semc).wait()

    issue_a(0, rows0, semA0)
    if ng > 1:
      issue_a(1, rows1, semA1)

    def body(p, carry):
      lg0 = 2 * p
      process(lg0, rows0, semA0, semC0)
      process(lg0 + 1, rows1, semA1, semC1)
      wait_c(lg0, rows0, semC0)

      @pl.when(lg0 + 2 < ng)
      def _():
        issue_a(lg0 + 2, rows0, semA0)

      wait_c(lg0 + 1, rows1, semC1)

      @pl.when(lg0 + 3 < ng)
      def _():
        issue_a(lg0 + 3, rows1, semA1)

      return carry

    lax.fori_loop(0, ng // 2, body, 0)
    if ng % 2:
      process(ng - 1, rows0, semA0, semC0)
      wait_c(ng - 1, rows0, semC0)

  return pl.kernel(
      body_fn,
      out_type=jax.ShapeDtypeStruct((NW * ng * GR, D), jnp.float32),
      mesh=_SC_MESH,
      scratch_types=[
          pltpu.VMEM((nch, CCH), jnp.int32),
          pltpu.VMEM((nch, CCH), jnp.int32),
          pltpu.VMEM((GR, D), jnp.float32),
          pltpu.VMEM((GR, D), jnp.float32),
          pltpu.SemaphoreType.DMA,
          pltpu.SemaphoreType.DMA,
          pltpu.SemaphoreType.DMA,
          pltpu.SemaphoreType.DMA,
          pltpu.SemaphoreType.DMA,
      ],
  )


_edge_combine0 = _make_edge_combine(NG0)
_edge_combine1 = _make_edge_combine(NG1)
E0 = NW * NG0 * GR
E1 = NW * NG1 * GR
_BME = 1600


def _layer1_body(agg_ref, deg_ref, w_ref, b_ref, out_ref):
  deg = jnp.maximum(deg_ref[0, :, 0:1] + deg_ref[1, :, 0:1], 1.0)
  agg = (agg_ref[0] + agg_ref[1]) / deg
  h = jnp.dot(agg, w_ref[...], preferred_element_type=jnp.float32) + b_ref[...]
  out_ref[...] = jnp.maximum(h, 0.0)


def _layer1(agg, degp, w1, b1):
  return pl.pallas_call(
      _layer1_body,
      out_shape=jax.ShapeDtypeStruct((NPAD, D), jnp.float32),
  )(agg, degp, w1, b1)


def _layer2_body(agg_ref, deg_ref, w2_ref, b2_ref, wt_ref, bt_ref, wb_ref,
                 a_ref, b_ref):
  deg = jnp.maximum(deg_ref[0, :, 0:1] + deg_ref[1, :, 0:1], 1.0)
  agg = (agg_ref[0] + agg_ref[1]) / deg
  h2 = jnp.dot(agg, w2_ref[...], preferred_element_type=jnp.float32) + b2_ref[...]
  a_ref[...] = jnp.dot(h2, wt_ref[...],
                       preferred_element_type=jnp.float32) + bt_ref[...]
  b_ref[...] = jnp.dot(h2, wb_ref[...], preferred_element_type=jnp.float32)


def _layer2(agg, degp, w2, b2, wm1t, bm1, wm1b):
  return pl.pallas_call(
      _layer2_body,
      out_shape=(jax.ShapeDtypeStruct((NPAD, D), jnp.float32),
                 jax.ShapeDtypeStruct((NPAD, D), jnp.float32)),
  )(agg, degp, w2, b2, wm1t, bm1, wm1b)


def _edge_mlp_body(c_ref, w_ref, b_ref, o_ref):
  c = jnp.maximum(c_ref[...], 0.0)
  o_ref[...] = jnp.dot(c, w_ref[...],
                       preferred_element_type=jnp.float32) + b_ref[...]


def _edge_mlp_slice0(c0, wm2, bm2):
  return pl.pallas_call(
      _edge_mlp_body,
      grid=(E0 // _BME,),
      in_specs=[
          pl.BlockSpec((_BME, D), lambda i: (i, 0)),
          pl.BlockSpec((D, D), lambda i: (0, 0)),
          pl.BlockSpec((1, D), lambda i: (0, 0)),
      ],
      out_specs=pl.BlockSpec((_BME, D), lambda i: (i, 0)),
      out_shape=jax.ShapeDtypeStruct((E, D), jnp.float32),
  )(c0, wm2, bm2)


def _edge_mlp_slice1_body(c_ref, w_ref, b_ref, prev_ref, o_ref):
  del prev_ref
  _edge_mlp_body(c_ref, w_ref, b_ref, o_ref)


def _edge_mlp_slice1(c1, wm2, bm2, prev):
  return pl.pallas_call(
      _edge_mlp_slice1_body,
      grid=(E1 // _BME,),
      in_specs=[
          pl.BlockSpec((_BME, D), lambda i: (i, 0)),
          pl.BlockSpec((D, D), lambda i: (0, 0)),
          pl.BlockSpec((1, D), lambda i: (0, 0)),
          pl.BlockSpec(memory_space=pl.ANY),
      ],
      out_specs=pl.BlockSpec((_BME, D), lambda i: (E0 // _BME + i, 0)),
      out_shape=jax.ShapeDtypeStruct((E, D), jnp.float32),
      input_output_aliases={3: 0},
  )(c1, wm2, bm2, prev)


def kernel(x, edge_index, W1, b1, W2, b2, Wm1, bm1, Wm2, bm2):
  srcs = edge_index[0].reshape(NW, NCHUNK, CHUNK)
  dsts = edge_index[1].reshape(NW, NCHUNK, CHUNK)
  srcs_c0 = edge_index[0, :E0].reshape(NW, NG0 * GRP, CCH)
  dsts_c0 = edge_index[1, :E0].reshape(NW, NG0 * GRP, CCH)
  srcs_c1 = edge_index[0, E0:].reshape(NW, NG1 * GRP, CCH)
  dsts_c1 = edge_index[1, E0:].reshape(NW, NG1 * GRP, CCH)
  zeros = jnp.zeros((NPAD, D), jnp.float32)
  zerosd = jnp.zeros((NPAD, DEGW), jnp.float32)
  ones = jnp.ones((CHUNK, DEGW), jnp.float32)

  degp = _degree(dsts, ones, zerosd)
  agg1 = _seg_sum(x, srcs, dsts, zeros)
  h = _layer1(agg1, degp, W1, b1.reshape(1, D))
  agg2 = _seg_sum(h, srcs, dsts, zeros)
  a_nodes, b_nodes = _layer2(agg2, degp, W2, b2.reshape(1, D),
                             Wm1[:D], bm1.reshape(1, D), Wm1[D:])
  c0 = _edge_combine0(a_nodes, b_nodes, srcs_c0, dsts_c0)
  out0 = _edge_mlp_slice0(c0, Wm2, bm2.reshape(1, D))
  c1 = _edge_combine1(a_nodes, b_nodes, srcs_c1, dsts_c1)
  return _edge_mlp_slice1(c1, Wm2, bm2.reshape(1, D), out0)


# speedup vs baseline: 1.0809x; 1.0809x over previous
import jax
import jax.numpy as jnp
from jax import lax
from jax.experimental import pallas as pl
from jax.experimental.pallas import tpu as pltpu
from jax.experimental.pallas import tpu_sc as plsc

N = 10000
E = 160000
D = 128
NC, NS = 2, 16
NW = NC * NS
EPW = E // NW
CHUNK = 125
NCHUNK = EPW // CHUNK
CCH = 40
NCCH = EPW // CCH
NPAD = 10240
RPT = NPAD // NS
DEGW = 128

_SC_MESH = plsc.VectorSubcoreMesh(
    core_axis_name="c", subcore_axis_name="s", num_cores=NC, num_subcores=NS)


def _seg_sum_body(x_hbm, srcs_hbm, dsts_hbm, zeros_hbm,
                  agg_hbm, idxs, idxd, rows0, rows1, acc, sem0, sem1):
  cid = lax.axis_index("c")
  sid = lax.axis_index("s")
  wid = cid * NS + sid
  pltpu.sync_copy(zeros_hbm.at[pl.ds(sid * RPT, RPT)],
                  acc.at[pl.ds(sid * RPT, RPT)])
  pltpu.sync_copy(srcs_hbm.at[wid], idxs)
  pltpu.sync_copy(dsts_hbm.at[wid], idxd)
  plsc.subcore_barrier()

  pltpu.async_copy(x_hbm.at[idxs.at[0]], rows0, sem0)
  pltpu.async_copy(x_hbm.at[idxs.at[1]], rows1, sem1)

  def body(p, carry):
    j0 = 2 * p
    pltpu.make_async_copy(x_hbm.at[idxs.at[j0]], rows0, sem0).wait()
    pltpu.sync_copy(rows0, acc.at[idxd.at[j0]], add=True)

    @pl.when(j0 + 2 < NCHUNK)
    def _():
      pltpu.async_copy(x_hbm.at[idxs.at[j0 + 2]], rows0, sem0)

    pltpu.make_async_copy(x_hbm.at[idxs.at[j0 + 1]], rows1, sem1).wait()
    pltpu.sync_copy(rows1, acc.at[idxd.at[j0 + 1]], add=True)

    @pl.when(j0 + 3 < NCHUNK)
    def _():
      pltpu.async_copy(x_hbm.at[idxs.at[j0 + 3]], rows1, sem1)

    return carry

  lax.fori_loop(0, NCHUNK // 2, body, 0)
  plsc.subcore_barrier()
  pltpu.sync_copy(acc.at[pl.ds(sid * RPT, RPT)],
                  agg_hbm.at[cid, pl.ds(sid * RPT, RPT)])


_seg_sum = pl.kernel(
    _seg_sum_body,
    out_type=jax.ShapeDtypeStruct((NC, NPAD, D), jnp.float32),
    mesh=_SC_MESH,
    scratch_types=[
        pltpu.VMEM((NCHUNK, CHUNK), jnp.int32),
        pltpu.VMEM((NCHUNK, CHUNK), jnp.int32),
        pltpu.VMEM((CHUNK, D), jnp.float32),
        pltpu.VMEM((CHUNK, D), jnp.float32),
        pltpu.VMEM_SHARED((NPAD, D), jnp.float32),
        pltpu.SemaphoreType.DMA,
        pltpu.SemaphoreType.DMA,
    ],
)


def _degree_body(dsts_hbm, ones_hbm, zerosd_hbm, deg_hbm,
                 idxd, ones_v, dacc, sem):
  cid = lax.axis_index("c")
  sid = lax.axis_index("s")
  wid = cid * NS + sid
  pltpu.sync_copy(zerosd_hbm.at[pl.ds(sid * RPT, RPT)],
                  dacc.at[pl.ds(sid * RPT, RPT)])
  pltpu.sync_copy(dsts_hbm.at[wid], idxd)
  pltpu.sync_copy(ones_hbm, ones_v)
  plsc.subcore_barrier()

  def body(j, carry):
    pltpu.async_copy(ones_v, dacc.at[idxd.at[j]], sem, add=True)
    return carry

  lax.fori_loop(0, NCHUNK, body, 0)

  def drain(j, carry):
    pltpu.make_async_copy(ones_v, dacc.at[idxd.at[j]], sem).wait()
    return carry

  lax.fori_loop(0, NCHUNK, drain, 0)
  plsc.subcore_barrier()
  pltpu.sync_copy(dacc.at[pl.ds(sid * RPT, RPT)],
                  deg_hbm.at[cid, pl.ds(sid * RPT, RPT)])


_degree = pl.kernel(
    _degree_body,
    out_type=jax.ShapeDtypeStruct((NC, NPAD, DEGW), jnp.float32),
    mesh=_SC_MESH,
    scratch_types=[
        pltpu.VMEM((NCHUNK, CHUNK), jnp.int32),
        pltpu.VMEM((CHUNK, DEGW), jnp.float32),
        pltpu.VMEM_SHARED((NPAD, DEGW), jnp.float32),
        pltpu.SemaphoreType.DMA,
    ],
)


GRP = 5
NG = NCCH // GRP
GR = GRP * CCH
NG0 = 12
NG1 = NG - NG0


def _make_edge_combine(ng):
  nch = ng * GRP

  def body_fn(a_hbm, b_hbm, srcs_hbm, dsts_hbm, c_hbm,
              idxs, idxd, rows0, rows1, semA0, semA1, semB):
    cid = lax.axis_index("c")
    sid = lax.axis_index("s")
    wid = cid * NS + sid
    pltpu.sync_copy(srcs_hbm.at[wid], idxs)
    pltpu.sync_copy(dsts_hbm.at[wid], idxd)
    base = wid * (ng * GR)

    def issue_a(lg, buf, sem):
      for k in range(GRP):
        pltpu.async_copy(a_hbm.at[idxs.at[lg * GRP + k]],
                         buf.at[pl.ds(k * CCH, CCH)], sem)

    def process(lg, buf, sem):
      for k in range(GRP):
        pltpu.make_async_copy(a_hbm.at[idxs.at[lg * GRP + k]],
                              buf.at[pl.ds(k * CCH, CCH)], sem).wait()
      descs = [
          pltpu.async_copy(b_hbm.at[idxd.at[lg * GRP + k]],
                           buf.at[pl.ds(k * CCH, CCH)], semB, add=True)
          for k in range(GRP)
      ]
      for desc in descs:
        desc.wait()
      pltpu.sync_copy(buf, c_hbm.at[pl.ds(base + lg * GR, GR)])

    issue_a(0, rows0, semA0)
    if ng > 1:
      issue_a(1, rows1, semA1)

    def body(p, carry):
      lg0 = 2 * p
      process(lg0, rows0, semA0)

      @pl.when(lg0 + 2 < ng)
      def _():
        issue_a(lg0 + 2, rows0, semA0)

      process(lg0 + 1, rows1, semA1)

      @pl.when(lg0 + 3 < ng)
      def _():
        issue_a(lg0 + 3, rows1, semA1)

      return carry

    lax.fori_loop(0, ng // 2, body, 0)
    if ng % 2:
      process(ng - 1, rows0, semA0)

  return pl.kernel(
      body_fn,
      out_type=jax.ShapeDtypeStruct((NW * ng * GR, D), jnp.float32),
      mesh=_SC_MESH,
      scratch_types=[
          pltpu.VMEM((nch, CCH), jnp.int32),
          pltpu.VMEM((nch, CCH), jnp.int32),
          pltpu.VMEM((GR, D), jnp.float32),
          pltpu.VMEM((GR, D), jnp.float32),
          pltpu.SemaphoreType.DMA,
          pltpu.SemaphoreType.DMA,
          pltpu.SemaphoreType.DMA,
      ],
  )


_edge_combine0 = _make_edge_combine(NG0)
_edge_combine1 = _make_edge_combine(NG1)
E0 = NW * NG0 * GR
E1 = NW * NG1 * GR
_BME = 1600


def _layer1_body(agg_ref, deg_ref, w_ref, b_ref, out_ref):
  deg = jnp.maximum(deg_ref[0, :, 0:1] + deg_ref[1, :, 0:1], 1.0)
  agg = (agg_ref[0] + agg_ref[1]) / deg
  h = jnp.dot(agg, w_ref[...], preferred_element_type=jnp.float32) + b_ref[...]
  out_ref[...] = jnp.maximum(h, 0.0)


def _layer1(agg, degp, w1, b1):
  return pl.pallas_call(
      _layer1_body,
      out_shape=jax.ShapeDtypeStruct((NPAD, D), jnp.float32),
  )(agg, degp, w1, b1)


def _layer2_body(agg_ref, deg_ref, w2_ref, b2_ref, wt_ref, bt_ref, wb_ref,
                 a_ref, b_ref):
  deg = jnp.maximum(deg_ref[0, :, 0:1] + deg_ref[1, :, 0:1], 1.0)
  agg = (agg_ref[0] + agg_ref[1]) / deg
  h2 = jnp.dot(agg, w2_ref[...], preferred_element_type=jnp.float32) + b2_ref[...]
  a_ref[...] = jnp.dot(h2, wt_ref[...],
                       preferred_element_type=jnp.float32) + bt_ref[...]
  b_ref[...] = jnp.dot(h2, wb_ref[...], preferred_element_type=jnp.float32)


def _layer2(agg, degp, w2, b2, wm1t, bm1, wm1b):
  return pl.pallas_call(
      _layer2_body,
      out_shape=(jax.ShapeDtypeStruct((NPAD, D), jnp.float32),
                 jax.ShapeDtypeStruct((NPAD, D), jnp.float32)),
  )(agg, degp, w2, b2, wm1t, bm1, wm1b)


def _edge_mlp_body(c_ref, w_ref, b_ref, o_ref):
  c = jnp.maximum(c_ref[...], 0.0)
  o_ref[...] = jnp.dot(c, w_ref[...],
                       preferred_element_type=jnp.float32) + b_ref[...]


def _edge_mlp_slice0(c0, wm2, bm2):
  return pl.pallas_call(
      _edge_mlp_body,
      grid=(E0 // _BME,),
      in_specs=[
          pl.BlockSpec((_BME, D), lambda i: (i, 0)),
          pl.BlockSpec((D, D), lambda i: (0, 0)),
          pl.BlockSpec((1, D), lambda i: (0, 0)),
      ],
      out_specs=pl.BlockSpec((_BME, D), lambda i: (i, 0)),
      out_shape=jax.ShapeDtypeStruct((E, D), jnp.float32),
  )(c0, wm2, bm2)


def _edge_mlp_slice1_body(c_ref, w_ref, b_ref, prev_ref, o_ref):
  del prev_ref
  _edge_mlp_body(c_ref, w_ref, b_ref, o_ref)


def _edge_mlp_slice1(c1, wm2, bm2, prev):
  return pl.pallas_call(
      _edge_mlp_slice1_body,
      grid=(E1 // _BME,),
      in_specs=[
          pl.BlockSpec((_BME, D), lambda i: (i, 0)),
          pl.BlockSpec((D, D), lambda i: (0, 0)),
          pl.BlockSpec((1, D), lambda i: (0, 0)),
          pl.BlockSpec(memory_space=pl.ANY),
      ],
      out_specs=pl.BlockSpec((_BME, D), lambda i: (E0 // _BME + i, 0)),
      out_shape=jax.ShapeDtypeStruct((E, D), jnp.float32),
      input_output_aliases={3: 0},
  )(c1, wm2, bm2, prev)


def kernel(x, edge_index, W1, b1, W2, b2, Wm1, bm1, Wm2, bm2):
  srcs = edge_index[0].reshape(NW, NCHUNK, CHUNK)
  dsts = edge_index[1].reshape(NW, NCHUNK, CHUNK)
  srcs_c0 = edge_index[0, :E0].reshape(NW, NG0 * GRP, CCH)
  dsts_c0 = edge_index[1, :E0].reshape(NW, NG0 * GRP, CCH)
  srcs_c1 = edge_index[0, E0:].reshape(NW, NG1 * GRP, CCH)
  dsts_c1 = edge_index[1, E0:].reshape(NW, NG1 * GRP, CCH)
  zeros = jnp.zeros((NPAD, D), jnp.float32)
  zerosd = jnp.zeros((NPAD, DEGW), jnp.float32)
  ones = jnp.ones((CHUNK, DEGW), jnp.float32)

  degp = _degree(dsts, ones, zerosd)
  agg1 = _seg_sum(x, srcs, dsts, zeros)
  h = _layer1(agg1, degp, W1, b1.reshape(1, D))
  agg2 = _seg_sum(h, srcs, dsts, zeros)
  a_nodes, b_nodes = _layer2(agg2, degp, W2, b2.reshape(1, D),
                             Wm1[:D], bm1.reshape(1, D), Wm1[D:])
  c0 = _edge_combine0(a_nodes, b_nodes, srcs_c0, dsts_c0)
  out0 = _edge_mlp_slice0(c0, Wm2, bm2.reshape(1, D))
  c1 = _edge_combine1(a_nodes, b_nodes, srcs_c1, dsts_c1)
  return _edge_mlp_slice1(c1, Wm2, bm2.reshape(1, D), out0)


# speedup vs baseline: 1.0845x; 1.0033x over previous
import jax
import jax.numpy as jnp
from jax import lax
from jax.experimental import pallas as pl
from jax.experimental.pallas import tpu as pltpu
from jax.experimental.pallas import tpu_sc as plsc

N = 10000
E = 160000
D = 128
NC, NS = 2, 16
NW = NC * NS
EPW = E // NW
CHUNK = 125
NCHUNK = EPW // CHUNK
CCH = 100
NCCH = EPW // CCH
NPAD = 10240
RPT = NPAD // NS
DEGW = 128

_SC_MESH = plsc.VectorSubcoreMesh(
    core_axis_name="c", subcore_axis_name="s", num_cores=NC, num_subcores=NS)


def _seg_sum_body(x_hbm, srcs_hbm, dsts_hbm, zeros_hbm,
                  agg_hbm, idxs, idxd, rows0, rows1, acc, sem0, sem1):
  cid = lax.axis_index("c")
  sid = lax.axis_index("s")
  wid = cid * NS + sid
  pltpu.sync_copy(zeros_hbm.at[pl.ds(sid * RPT, RPT)],
                  acc.at[pl.ds(sid * RPT, RPT)])
  pltpu.sync_copy(srcs_hbm.at[wid], idxs)
  pltpu.sync_copy(dsts_hbm.at[wid], idxd)
  plsc.subcore_barrier()

  pltpu.async_copy(x_hbm.at[idxs.at[0]], rows0, sem0)
  pltpu.async_copy(x_hbm.at[idxs.at[1]], rows1, sem1)

  def body(p, carry):
    j0 = 2 * p
    pltpu.make_async_copy(x_hbm.at[idxs.at[j0]], rows0, sem0).wait()
    pltpu.sync_copy(rows0, acc.at[idxd.at[j0]], add=True)

    @pl.when(j0 + 2 < NCHUNK)
    def _():
      pltpu.async_copy(x_hbm.at[idxs.at[j0 + 2]], rows0, sem0)

    pltpu.make_async_copy(x_hbm.at[idxs.at[j0 + 1]], rows1, sem1).wait()
    pltpu.sync_copy(rows1, acc.at[idxd.at[j0 + 1]], add=True)

    @pl.when(j0 + 3 < NCHUNK)
    def _():
      pltpu.async_copy(x_hbm.at[idxs.at[j0 + 3]], rows1, sem1)

    return carry

  lax.fori_loop(0, NCHUNK // 2, body, 0)
  plsc.subcore_barrier()
  pltpu.sync_copy(acc.at[pl.ds(sid * RPT, RPT)],
                  agg_hbm.at[cid, pl.ds(sid * RPT, RPT)])


_seg_sum = pl.kernel(
    _seg_sum_body,
    out_type=jax.ShapeDtypeStruct((NC, NPAD, D), jnp.float32),
    mesh=_SC_MESH,
    scratch_types=[
        pltpu.VMEM((NCHUNK, CHUNK), jnp.int32),
        pltpu.VMEM((NCHUNK, CHUNK), jnp.int32),
        pltpu.VMEM((CHUNK, D), jnp.float32),
        pltpu.VMEM((CHUNK, D), jnp.float32),
        pltpu.VMEM_SHARED((NPAD, D), jnp.float32),
        pltpu.SemaphoreType.DMA,
        pltpu.SemaphoreType.DMA,
    ],
)


def _degree_body(dsts_hbm, ones_hbm, zerosd_hbm, deg_hbm,
                 idxd, ones_v, dacc, sem):
  cid = lax.axis_index("c")
  sid = lax.axis_index("s")
  wid = cid * NS + sid
  pltpu.sync_copy(zerosd_hbm.at[pl.ds(sid * RPT, RPT)],
                  dacc.at[pl.ds(sid * RPT, RPT)])
  pltpu.sync_copy(dsts_hbm.at[wid], idxd)
  pltpu.sync_copy(ones_hbm, ones_v)
  plsc.subcore_barrier()

  def body(j, carry):
    pltpu.async_copy(ones_v, dacc.at[idxd.at[j]], sem, add=True)
    return carry

  lax.fori_loop(0, NCHUNK, body, 0)

  def drain(j, carry):
    pltpu.make_async_copy(ones_v, dacc.at[idxd.at[j]], sem).wait()
    return carry

  lax.fori_loop(0, NCHUNK, drain, 0)
  plsc.subcore_barrier()
  pltpu.sync_copy(dacc.at[pl.ds(sid * RPT, RPT)],
                  deg_hbm.at[cid, pl.ds(sid * RPT, RPT)])


_degree = pl.kernel(
    _degree_body,
    out_type=jax.ShapeDtypeStruct((NC, NPAD, DEGW), jnp.float32),
    mesh=_SC_MESH,
    scratch_types=[
        pltpu.VMEM((NCHUNK, CHUNK), jnp.int32),
        pltpu.VMEM((CHUNK, DEGW), jnp.float32),
        pltpu.VMEM_SHARED((NPAD, DEGW), jnp.float32),
        pltpu.SemaphoreType.DMA,
    ],
)


GRP = 2
NG = NCCH // GRP
GR = GRP * CCH
NG0 = 12
NG1 = NG - NG0


def _make_edge_combine(ng):
  nch = ng * GRP

  def body_fn(a_hbm, b_hbm, srcs_hbm, dsts_hbm, c_hbm,
              idxs, idxd, rows0, rows1, semA0, semA1, semB):
    cid = lax.axis_index("c")
    sid = lax.axis_index("s")
    wid = cid * NS + sid
    pltpu.sync_copy(srcs_hbm.at[wid], idxs)
    pltpu.sync_copy(dsts_hbm.at[wid], idxd)
    base = wid * (ng * GR)

    def issue_a(lg, buf, sem):
      for k in range(GRP):
        pltpu.async_copy(a_hbm.at[idxs.at[lg * GRP + k]],
                         buf.at[pl.ds(k * CCH, CCH)], sem)

    def process(lg, buf, sem):
      for k in range(GRP):
        pltpu.make_async_copy(a_hbm.at[idxs.at[lg * GRP + k]],
                              buf.at[pl.ds(k * CCH, CCH)], sem).wait()
      descs = [
          pltpu.async_copy(b_hbm.at[idxd.at[lg * GRP + k]],
                           buf.at[pl.ds(k * CCH, CCH)], semB, add=True)
          for k in range(GRP)
      ]
      for desc in descs:
        desc.wait()
      pltpu.sync_copy(buf, c_hbm.at[pl.ds(base + lg * GR, GR)])

    issue_a(0, rows0, semA0)
    if ng > 1:
      issue_a(1, rows1, semA1)

    def body(p, carry):
      lg0 = 2 * p
      process(lg0, rows0, semA0)

      @pl.when(lg0 + 2 < ng)
      def _():
        issue_a(lg0 + 2, rows0, semA0)

      process(lg0 + 1, rows1, semA1)

      @pl.when(lg0 + 3 < ng)
      def _():
        issue_a(lg0 + 3, rows1, semA1)

      return carry

    lax.fori_loop(0, ng // 2, body, 0)
    if ng % 2:
      process(ng - 1, rows0, semA0)

  return pl.kernel(
      body_fn,
      out_type=jax.ShapeDtypeStruct((NW * ng * GR, D), jnp.float32),
      mesh=_SC_MESH,
      scratch_types=[
          pltpu.VMEM((nch, CCH), jnp.int32),
          pltpu.VMEM((nch, CCH), jnp.int32),
          pltpu.VMEM((GR, D), jnp.float32),
          pltpu.VMEM((GR, D), jnp.float32),
          pltpu.SemaphoreType.DMA,
          pltpu.SemaphoreType.DMA,
          pltpu.SemaphoreType.DMA,
      ],
  )


_edge_combine0 = _make_edge_combine(NG0)
_edge_combine1 = _make_edge_combine(NG1)
E0 = NW * NG0 * GR
E1 = NW * NG1 * GR
_BME = 1600


def _layer1_body(agg_ref, deg_ref, w_ref, b_ref, out_ref):
  deg = jnp.maximum(deg_ref[0, :, 0:1] + deg_ref[1, :, 0:1], 1.0)
  agg = (agg_ref[0] + agg_ref[1]) / deg
  h = jnp.dot(agg, w_ref[...], preferred_element_type=jnp.float32) + b_ref[...]
  out_ref[...] = jnp.maximum(h, 0.0)


def _layer1(agg, degp, w1, b1):
  return pl.pallas_call(
      _layer1_body,
      out_shape=jax.ShapeDtypeStruct((NPAD, D), jnp.float32),
  )(agg, degp, w1, b1)


def _layer2_body(agg_ref, deg_ref, w2_ref, b2_ref, wt_ref, bt_ref, wb_ref,
                 a_ref, b_ref):
  deg = jnp.maximum(deg_ref[0, :, 0:1] + deg_ref[1, :, 0:1], 1.0)
  agg = (agg_ref[0] + agg_ref[1]) / deg
  h2 = jnp.dot(agg, w2_ref[...], preferred_element_type=jnp.float32) + b2_ref[...]
  a_ref[...] = jnp.dot(h2, wt_ref[...],
                       preferred_element_type=jnp.float32) + bt_ref[...]
  b_ref[...] = jnp.dot(h2, wb_ref[...], preferred_element_type=jnp.float32)


def _layer2(agg, degp, w2, b2, wm1t, bm1, wm1b):
  return pl.pallas_call(
      _layer2_body,
      out_shape=(jax.ShapeDtypeStruct((NPAD, D), jnp.float32),
                 jax.ShapeDtypeStruct((NPAD, D), jnp.float32)),
  )(agg, degp, w2, b2, wm1t, bm1, wm1b)


def _edge_mlp_body(c_ref, w_ref, b_ref, o_ref):
  c = jnp.maximum(c_ref[...], 0.0)
  o_ref[...] = jnp.dot(c, w_ref[...],
                       preferred_element_type=jnp.float32) + b_ref[...]


def _edge_mlp_slice0(c0, wm2, bm2):
  return pl.pallas_call(
      _edge_mlp_body,
      grid=(E0 // _BME,),
      in_specs=[
          pl.BlockSpec((_BME, D), lambda i: (i, 0)),
          pl.BlockSpec((D, D), lambda i: (0, 0)),
          pl.BlockSpec((1, D), lambda i: (0, 0)),
      ],
      out_specs=pl.BlockSpec((_BME, D), lambda i: (i, 0)),
      out_shape=jax.ShapeDtypeStruct((E, D), jnp.float32),
  )(c0, wm2, bm2)


def _edge_mlp_slice1_body(c_ref, w_ref, b_ref, prev_ref, o_ref):
  del prev_ref
  _edge_mlp_body(c_ref, w_ref, b_ref, o_ref)


def _edge_mlp_slice1(c1, wm2, bm2, prev):
  return pl.pallas_call(
      _edge_mlp_slice1_body,
      grid=(E1 // _BME,),
      in_specs=[
          pl.BlockSpec((_BME, D), lambda i: (i, 0)),
          pl.BlockSpec((D, D), lambda i: (0, 0)),
          pl.BlockSpec((1, D), lambda i: (0, 0)),
          pl.BlockSpec(memory_space=pl.ANY),
      ],
      out_specs=pl.BlockSpec((_BME, D), lambda i: (E0 // _BME + i, 0)),
      out_shape=jax.ShapeDtypeStruct((E, D), jnp.float32),
      input_output_aliases={3: 0},
  )(c1, wm2, bm2, prev)


def kernel(x, edge_index, W1, b1, W2, b2, Wm1, bm1, Wm2, bm2):
  srcs = edge_index[0].reshape(NW, NCHUNK, CHUNK)
  dsts = edge_index[1].reshape(NW, NCHUNK, CHUNK)
  srcs_c0 = edge_index[0, :E0].reshape(NW, NG0 * GRP, CCH)
  dsts_c0 = edge_index[1, :E0].reshape(NW, NG0 * GRP, CCH)
  srcs_c1 = edge_index[0, E0:].reshape(NW, NG1 * GRP, CCH)
  dsts_c1 = edge_index[1, E0:].reshape(NW, NG1 * GRP, CCH)
  zeros = jnp.zeros((NPAD, D), jnp.float32)
  zerosd = jnp.zeros((NPAD, DEGW), jnp.float32)
  ones = jnp.ones((CHUNK, DEGW), jnp.float32)

  degp = _degree(dsts, ones, zerosd)
  agg1 = _seg_sum(x, srcs, dsts, zeros)
  h = _layer1(agg1, degp, W1, b1.reshape(1, D))
  agg2 = _seg_sum(h, srcs, dsts, zeros)
  a_nodes, b_nodes = _layer2(agg2, degp, W2, b2.reshape(1, D),
                             Wm1[:D], bm1.reshape(1, D), Wm1[D:])
  c0 = _edge_combine0(a_nodes, b_nodes, srcs_c0, dsts_c0)
  out0 = _edge_mlp_slice0(c0, Wm2, bm2.reshape(1, D))
  c1 = _edge_combine1(a_nodes, b_nodes, srcs_c1, dsts_c1)
  return _edge_mlp_slice1(c1, Wm2, bm2.reshape(1, D), out0)


# speedup vs baseline: 1.0914x; 1.0064x over previous
import jax
import jax.numpy as jnp
from jax import lax
from jax.experimental import pallas as pl
from jax.experimental.pallas import tpu as pltpu
from jax.experimental.pallas import tpu_sc as plsc

N = 10000
E = 160000
D = 128
NC, NS = 2, 16
NW = NC * NS
EPW = E // NW
CHUNK = 125
NCHUNK = EPW // CHUNK
CCH = 40
NCCH = EPW // CCH
NPAD = 10240
RPT = NPAD // NS
DEGW = 128

_SC_MESH = plsc.VectorSubcoreMesh(
    core_axis_name="c", subcore_axis_name="s", num_cores=NC, num_subcores=NS)


def _seg_sum_body(x_hbm, srcs_hbm, dsts_hbm, zeros_hbm,
                  agg_hbm, idxs, idxd, rows0, rows1, acc, sem0, sem1):
  cid = lax.axis_index("c")
  sid = lax.axis_index("s")
  wid = cid * NS + sid
  pltpu.sync_copy(zeros_hbm.at[pl.ds(sid * RPT, RPT)],
                  acc.at[pl.ds(sid * RPT, RPT)])
  pltpu.sync_copy(srcs_hbm.at[wid], idxs)
  pltpu.sync_copy(dsts_hbm.at[wid], idxd)
  plsc.subcore_barrier()

  pltpu.async_copy(x_hbm.at[idxs.at[0]], rows0, sem0)
  pltpu.async_copy(x_hbm.at[idxs.at[1]], rows1, sem1)

  def body(p, carry):
    j0 = 2 * p
    pltpu.make_async_copy(x_hbm.at[idxs.at[j0]], rows0, sem0).wait()
    pltpu.sync_copy(rows0, acc.at[idxd.at[j0]], add=True)

    @pl.when(j0 + 2 < NCHUNK)
    def _():
      pltpu.async_copy(x_hbm.at[idxs.at[j0 + 2]], rows0, sem0)

    pltpu.make_async_copy(x_hbm.at[idxs.at[j0 + 1]], rows1, sem1).wait()
    pltpu.sync_copy(rows1, acc.at[idxd.at[j0 + 1]], add=True)

    @pl.when(j0 + 3 < NCHUNK)
    def _():
      pltpu.async_copy(x_hbm.at[idxs.at[j0 + 3]], rows1, sem1)

    return carry

  lax.fori_loop(0, NCHUNK // 2, body, 0)
  plsc.subcore_barrier()
  pltpu.sync_copy(acc.at[pl.ds(sid * RPT, RPT)],
                  agg_hbm.at[cid, pl.ds(sid * RPT, RPT)])


_seg_sum = pl.kernel(
    _seg_sum_body,
    out_type=jax.ShapeDtypeStruct((NC, NPAD, D), jnp.float32),
    mesh=_SC_MESH,
    scratch_types=[
        pltpu.VMEM((NCHUNK, CHUNK), jnp.int32),
        pltpu.VMEM((NCHUNK, CHUNK), jnp.int32),
        pltpu.VMEM((CHUNK, D), jnp.float32),
        pltpu.VMEM((CHUNK, D), jnp.float32),
        pltpu.VMEM_SHARED((NPAD, D), jnp.float32),
        pltpu.SemaphoreType.DMA,
        pltpu.SemaphoreType.DMA,
    ],
)


def _degree_body(dsts_hbm, ones_hbm, zerosd_hbm, deg_hbm,
                 idxd, ones_v, dacc, sem):
  cid = lax.axis_index("c")
  sid = lax.axis_index("s")
  wid = cid * NS + sid
  pltpu.sync_copy(zerosd_hbm.at[pl.ds(sid * RPT, RPT)],
                  dacc.at[pl.ds(sid * RPT, RPT)])
  pltpu.sync_copy(dsts_hbm.at[wid], idxd)
  pltpu.sync_copy(ones_hbm, ones_v)
  plsc.subcore_barrier()

  def body(j, carry):
    pltpu.async_copy(ones_v, dacc.at[idxd.at[j]], sem, add=True)
    return carry

  lax.fori_loop(0, NCHUNK, body, 0)

  def drain(j, carry):
    pltpu.make_async_copy(ones_v, dacc.at[idxd.at[j]], sem).wait()
    return carry

  lax.fori_loop(0, NCHUNK, drain, 0)
  plsc.subcore_barrier()
  pltpu.sync_copy(dacc.at[pl.ds(sid * RPT, RPT)],
                  deg_hbm.at[cid, pl.ds(sid * RPT, RPT)])


_degree = pl.kernel(
    _degree_body,
    out_type=jax.ShapeDtypeStruct((NC, NPAD, DEGW), jnp.float32),
    mesh=_SC_MESH,
    scratch_types=[
        pltpu.VMEM((NCHUNK, CHUNK), jnp.int32),
        pltpu.VMEM((CHUNK, DEGW), jnp.float32),
        pltpu.VMEM_SHARED((NPAD, DEGW), jnp.float32),
        pltpu.SemaphoreType.DMA,
    ],
)


GRP = 5
NG = NCCH // GRP
GR = GRP * CCH
NGS = (10, 10, 5)


def _make_edge_combine(ng):
  nch = ng * GRP

  def body_fn(a_hbm, b_hbm, srcs_hbm, dsts_hbm, c_hbm,
              idxs, idxd, rows0, rows1, semA0, semA1, semB):
    cid = lax.axis_index("c")
    sid = lax.axis_index("s")
    wid = cid * NS + sid
    pltpu.sync_copy(srcs_hbm.at[wid], idxs)
    pltpu.sync_copy(dsts_hbm.at[wid], idxd)
    base = wid * (ng * GR)

    def issue_a(lg, buf, sem):
      for k in range(GRP):
        pltpu.async_copy(a_hbm.at[idxs.at[lg * GRP + k]],
                         buf.at[pl.ds(k * CCH, CCH)], sem)

    def process(lg, buf, sem):
      for k in range(GRP):
        pltpu.make_async_copy(a_hbm.at[idxs.at[lg * GRP + k]],
                              buf.at[pl.ds(k * CCH, CCH)], sem).wait()
      descs = [
          pltpu.async_copy(b_hbm.at[idxd.at[lg * GRP + k]],
                           buf.at[pl.ds(k * CCH, CCH)], semB, add=True)
          for k in range(GRP)
      ]
      for desc in descs:
        desc.wait()
      pltpu.sync_copy(buf, c_hbm.at[pl.ds(base + lg * GR, GR)])

    issue_a(0, rows0, semA0)
    if ng > 1:
      issue_a(1, rows1, semA1)

    def body(p, carry):
      lg0 = 2 * p
      process(lg0, rows0, semA0)

      @pl.when(lg0 + 2 < ng)
      def _():
        issue_a(lg0 + 2, rows0, semA0)

      process(lg0 + 1, rows1, semA1)

      @pl.when(lg0 + 3 < ng)
      def _():
        issue_a(lg0 + 3, rows1, semA1)

      return carry

    lax.fori_loop(0, ng // 2, body, 0)
    if ng % 2:
      process(ng - 1, rows0, semA0)

  return pl.kernel(
      body_fn,
      out_type=jax.ShapeDtypeStruct((NW * ng * GR, D), jnp.float32),
      mesh=_SC_MESH,
      scratch_types=[
          pltpu.VMEM((nch, CCH), jnp.int32),
          pltpu.VMEM((nch, CCH), jnp.int32),
          pltpu.VMEM((GR, D), jnp.float32),
          pltpu.VMEM((GR, D), jnp.float32),
          pltpu.SemaphoreType.DMA,
          pltpu.SemaphoreType.DMA,
          pltpu.SemaphoreType.DMA,
      ],
  )


_combine_by_ng = {ng: _make_edge_combine(ng) for ng in sorted(set(NGS))}
ES = tuple(NW * ng * GR for ng in NGS)
_BME = 1600


def _layer1_body(agg_ref, deg_ref, w_ref, b_ref, out_ref):
  deg = jnp.maximum(deg_ref[0, :, 0:1] + deg_ref[1, :, 0:1], 1.0)
  agg = (agg_ref[0] + agg_ref[1]) / deg
  h = jnp.dot(agg, w_ref[...], preferred_element_type=jnp.float32) + b_ref[...]
  out_ref[...] = jnp.maximum(h, 0.0)


def _layer1(agg, degp, w1, b1):
  return pl.pallas_call(
      _layer1_body,
      out_shape=jax.ShapeDtypeStruct((NPAD, D), jnp.float32),
  )(agg, degp, w1, b1)


def _layer2_body(agg_ref, deg_ref, w2_ref, b2_ref, wt_ref, bt_ref, wb_ref,
                 a_ref, b_ref):
  deg = jnp.maximum(deg_ref[0, :, 0:1] + deg_ref[1, :, 0:1], 1.0)
  agg = (agg_ref[0] + agg_ref[1]) / deg
  h2 = jnp.dot(agg, w2_ref[...], preferred_element_type=jnp.float32) + b2_ref[...]
  a_ref[...] = jnp.dot(h2, wt_ref[...],
                       preferred_element_type=jnp.float32) + bt_ref[...]
  b_ref[...] = jnp.dot(h2, wb_ref[...], preferred_element_type=jnp.float32)


def _layer2(agg, degp, w2, b2, wm1t, bm1, wm1b):
  return pl.pallas_call(
      _layer2_body,
      out_shape=(jax.ShapeDtypeStruct((NPAD, D), jnp.float32),
                 jax.ShapeDtypeStruct((NPAD, D), jnp.float32)),
  )(agg, degp, w2, b2, wm1t, bm1, wm1b)


def _make_edge_mlp(n_rows, row_off, aliased):
  boff = row_off // _BME

  def body(c_ref, w_ref, b_ref, *refs):
    o_ref = refs[-1]
    c = jnp.maximum(c_ref[...], 0.0)
    o_ref[...] = jnp.dot(c, w_ref[...],
                         preferred_element_type=jnp.float32) + b_ref[...]

  in_specs = [
      pl.BlockSpec((_BME, D), lambda i: (i, 0)),
      pl.BlockSpec((D, D), lambda i: (0, 0)),
      pl.BlockSpec((1, D), lambda i: (0, 0)),
  ]
  kwargs = {}
  if aliased:
    in_specs.append(pl.BlockSpec(memory_space=pl.ANY))
    kwargs["input_output_aliases"] = {3: 0}

  return pl.pallas_call(
      body,
      grid=(n_rows // _BME,),
      in_specs=in_specs,
      out_specs=pl.BlockSpec((_BME, D), lambda i: (boff + i, 0)),
      out_shape=jax.ShapeDtypeStruct((E, D), jnp.float32),
      **kwargs,
  )


_edge_mlps = []
_off = 0
for _i, _es in enumerate(ES):
  _edge_mlps.append(_make_edge_mlp(_es, _off, aliased=_i > 0))
  _off += _es


def kernel(x, edge_index, W1, b1, W2, b2, Wm1, bm1, Wm2, bm2):
  srcs = edge_index[0].reshape(NW, NCHUNK, CHUNK)
  dsts = edge_index[1].reshape(NW, NCHUNK, CHUNK)
  zeros = jnp.zeros((NPAD, D), jnp.float32)
  zerosd = jnp.zeros((NPAD, DEGW), jnp.float32)
  ones = jnp.ones((CHUNK, DEGW), jnp.float32)

  degp = _degree(dsts, ones, zerosd)
  agg1 = _seg_sum(x, srcs, dsts, zeros)
  h = _layer1(agg1, degp, W1, b1.reshape(1, D))
  agg2 = _seg_sum(h, srcs, dsts, zeros)
  a_nodes, b_nodes = _layer2(agg2, degp, W2, b2.reshape(1, D),
                             Wm1[:D], bm1.reshape(1, D), Wm1[D:])

  bm2r = bm2.reshape(1, D)
  out = None
  off = 0
  for i, (ng, es) in enumerate(zip(NGS, ES)):
    sl_src = edge_index[0, off:off + es].reshape(NW, ng * GRP, CCH)
    sl_dst = edge_index[1, off:off + es].reshape(NW, ng * GRP, CCH)
    c = _combine_by_ng[ng](a_nodes, b_nodes, sl_src, sl_dst)
    if i == 0:
      out = _edge_mlps[i](c, Wm2, bm2r)
    else:
      out = _edge_mlps[i](c, Wm2, bm2r, out)
    off += es
  return out


# speedup vs baseline: 1.0971x; 1.0052x over previous
import jax
import jax.numpy as jnp
from jax import lax
from jax.experimental import pallas as pl
from jax.experimental.pallas import tpu as pltpu
from jax.experimental.pallas import tpu_sc as plsc

N = 10000
E = 160000
D = 128
NC, NS = 2, 16
NW = NC * NS
EPW = E // NW
CHUNK = 125
NCHUNK = EPW // CHUNK
CCH = 40
NCCH = EPW // CCH
NPAD = 10240
RPT = NPAD // NS
DEGW = 128

_SC_MESH = plsc.VectorSubcoreMesh(
    core_axis_name="c", subcore_axis_name="s", num_cores=NC, num_subcores=NS)


def _seg_sum_body(x_hbm, srcs_hbm, dsts_hbm, zeros_hbm,
                  agg_hbm, idxs, idxd, rows0, rows1, acc, sem0, sem1):
  cid = lax.axis_index("c")
  sid = lax.axis_index("s")
  wid = cid * NS + sid
  pltpu.async_copy(zeros_hbm.at[pl.ds(sid * RPT, RPT)],
                   acc.at[pl.ds(sid * RPT, RPT)], sem0)
  pltpu.async_copy(srcs_hbm.at[wid], idxs, sem0)
  pltpu.async_copy(dsts_hbm.at[wid], idxd, sem0)
  pltpu.make_async_copy(zeros_hbm.at[pl.ds(sid * RPT, RPT)],
                        acc.at[pl.ds(sid * RPT, RPT)], sem0).wait()
  pltpu.make_async_copy(srcs_hbm.at[wid], idxs, sem0).wait()
  pltpu.make_async_copy(dsts_hbm.at[wid], idxd, sem0).wait()
  plsc.subcore_barrier()

  pltpu.async_copy(x_hbm.at[idxs.at[0]], rows0, sem0)
  pltpu.async_copy(x_hbm.at[idxs.at[1]], rows1, sem1)

  def body(p, carry):
    j0 = 2 * p
    pltpu.make_async_copy(x_hbm.at[idxs.at[j0]], rows0, sem0).wait()
    pltpu.sync_copy(rows0, acc.at[idxd.at[j0]], add=True)

    @pl.when(j0 + 2 < NCHUNK)
    def _():
      pltpu.async_copy(x_hbm.at[idxs.at[j0 + 2]], rows0, sem0)

    pltpu.make_async_copy(x_hbm.at[idxs.at[j0 + 1]], rows1, sem1).wait()
    pltpu.sync_copy(rows1, acc.at[idxd.at[j0 + 1]], add=True)

    @pl.when(j0 + 3 < NCHUNK)
    def _():
      pltpu.async_copy(x_hbm.at[idxs.at[j0 + 3]], rows1, sem1)

    return carry

  lax.fori_loop(0, NCHUNK // 2, body, 0)
  plsc.subcore_barrier()
  pltpu.sync_copy(acc.at[pl.ds(sid * RPT, RPT)],
                  agg_hbm.at[cid, pl.ds(sid * RPT, RPT)])


_seg_sum = pl.kernel(
    _seg_sum_body,
    out_type=jax.ShapeDtypeStruct((NC, NPAD, D), jnp.float32),
    mesh=_SC_MESH,
    scratch_types=[
        pltpu.VMEM((NCHUNK, CHUNK), jnp.int32),
        pltpu.VMEM((NCHUNK, CHUNK), jnp.int32),
        pltpu.VMEM((CHUNK, D), jnp.float32),
        pltpu.VMEM((CHUNK, D), jnp.float32),
        pltpu.VMEM_SHARED((NPAD, D), jnp.float32),
        pltpu.SemaphoreType.DMA,
        pltpu.SemaphoreType.DMA,
    ],
)


def _degree_body(dsts_hbm, ones_hbm, zerosd_hbm, deg_hbm,
                 idxd, ones_v, dacc, sem):
  cid = lax.axis_index("c")
  sid = lax.axis_index("s")
  wid = cid * NS + sid
  pltpu.sync_copy(zerosd_hbm.at[pl.ds(sid * RPT, RPT)],
                  dacc.at[pl.ds(sid * RPT, RPT)])
  pltpu.sync_copy(dsts_hbm.at[wid], idxd)
  pltpu.sync_copy(ones_hbm, ones_v)
  plsc.subcore_barrier()

  def body(j, carry):
    pltpu.async_copy(ones_v, dacc.at[idxd.at[j]], sem, add=True)
    return carry

  lax.fori_loop(0, NCHUNK, body, 0)

  def drain(j, carry):
    pltpu.make_async_copy(ones_v, dacc.at[idxd.at[j]], sem).wait()
    return carry

  lax.fori_loop(0, NCHUNK, drain, 0)
  plsc.subcore_barrier()
  pltpu.sync_copy(dacc.at[pl.ds(sid * RPT, RPT)],
                  deg_hbm.at[cid, pl.ds(sid * RPT, RPT)])


_degree = pl.kernel(
    _degree_body,
    out_type=jax.ShapeDtypeStruct((NC, NPAD, DEGW), jnp.float32),
    mesh=_SC_MESH,
    scratch_types=[
        pltpu.VMEM((NCHUNK, CHUNK), jnp.int32),
        pltpu.VMEM((CHUNK, DEGW), jnp.float32),
        pltpu.VMEM_SHARED((NPAD, DEGW), jnp.float32),
        pltpu.SemaphoreType.DMA,
    ],
)


GRP = 5
NG = NCCH // GRP
GR = GRP * CCH
NGS = (10, 10, 5)


def _make_edge_combine(ng):
  nch = ng * GRP

  def body_fn(a_hbm, b_hbm, srcs_hbm, dsts_hbm, c_hbm,
              idxs, idxd, rows0, rows1, semA0, semA1, semB):
    cid = lax.axis_index("c")
    sid = lax.axis_index("s")
    wid = cid * NS + sid
    pltpu.sync_copy(srcs_hbm.at[wid], idxs)
    pltpu.sync_copy(dsts_hbm.at[wid], idxd)
    base = wid * (ng * GR)

    def issue_a(lg, buf, sem):
      for k in range(GRP):
        pltpu.async_copy(a_hbm.at[idxs.at[lg * GRP + k]],
                         buf.at[pl.ds(k * CCH, CCH)], sem)

    def process(lg, buf, sem):
      for k in range(GRP):
        pltpu.make_async_copy(a_hbm.at[idxs.at[lg * GRP + k]],
                              buf.at[pl.ds(k * CCH, CCH)], sem).wait()
      descs = [
          pltpu.async_copy(b_hbm.at[idxd.at[lg * GRP + k]],
                           buf.at[pl.ds(k * CCH, CCH)], semB, add=True)
          for k in range(GRP)
      ]
      for desc in descs:
        desc.wait()
      pltpu.sync_copy(buf, c_hbm.at[pl.ds(base + lg * GR, GR)])

    issue_a(0, rows0, semA0)
    if ng > 1:
      issue_a(1, rows1, semA1)

    def body(p, carry):
      lg0 = 2 * p
      process(lg0, rows0, semA0)

      @pl.when(lg0 + 2 < ng)
      def _():
        issue_a(lg0 + 2, rows0, semA0)

      process(lg0 + 1, rows1, semA1)

      @pl.when(lg0 + 3 < ng)
      def _():
        issue_a(lg0 + 3, rows1, semA1)

      return carry

    lax.fori_loop(0, ng // 2, body, 0)
    if ng % 2:
      process(ng - 1, rows0, semA0)

  return pl.kernel(
      body_fn,
      out_type=jax.ShapeDtypeStruct((NW * ng * GR, D), jnp.float32),
      mesh=_SC_MESH,
      scratch_types=[
          pltpu.VMEM((nch, CCH), jnp.int32),
          pltpu.VMEM((nch, CCH), jnp.int32),
          pltpu.VMEM((GR, D), jnp.float32),
          pltpu.VMEM((GR, D), jnp.float32),
          pltpu.SemaphoreType.DMA,
          pltpu.SemaphoreType.DMA,
          pltpu.SemaphoreType.DMA,
      ],
  )


_combine_by_ng = {ng: _make_edge_combine(ng) for ng in sorted(set(NGS))}
ES = tuple(NW * ng * GR for ng in NGS)
_BME = 1600


def _layer1_body(agg_ref, deg_ref, w_ref, b_ref, out_ref):
  deg = jnp.maximum(deg_ref[0, :, 0:1] + deg_ref[1, :, 0:1], 1.0)
  agg = (agg_ref[0] + agg_ref[1]) / deg
  h = jnp.dot(agg, w_ref[...], preferred_element_type=jnp.float32) + b_ref[...]
  out_ref[...] = jnp.maximum(h, 0.0)


def _layer1(agg, degp, w1, b1):
  return pl.pallas_call(
      _layer1_body,
      out_shape=jax.ShapeDtypeStruct((NPAD, D), jnp.float32),
  )(agg, degp, w1, b1)


def _layer2_body(agg_ref, deg_ref, w2_ref, b2_ref, wt_ref, bt_ref, wb_ref,
                 a_ref, b_ref):
  deg = jnp.maximum(deg_ref[0, :, 0:1] + deg_ref[1, :, 0:1], 1.0)
  agg = (agg_ref[0] + agg_ref[1]) / deg
  h2 = jnp.dot(agg, w2_ref[...], preferred_element_type=jnp.float32) + b2_ref[...]
  a_ref[...] = jnp.dot(h2, wt_ref[...],
                       preferred_element_type=jnp.float32) + bt_ref[...]
  b_ref[...] = jnp.dot(h2, wb_ref[...], preferred_element_type=jnp.float32)


def _layer2(agg, degp, w2, b2, wm1t, bm1, wm1b):
  return pl.pallas_call(
      _layer2_body,
      out_shape=(jax.ShapeDtypeStruct((NPAD, D), jnp.float32),
                 jax.ShapeDtypeStruct((NPAD, D), jnp.float32)),
  )(agg, degp, w2, b2, wm1t, bm1, wm1b)


def _make_edge_mlp(n_rows, row_off, aliased):
  boff = row_off // _BME

  def body(c_ref, w_ref, b_ref, *refs):
    o_ref = refs[-1]
    c = jnp.maximum(c_ref[...], 0.0)
    o_ref[...] = jnp.dot(c, w_ref[...],
                         preferred_element_type=jnp.float32) + b_ref[...]

  in_specs = [
      pl.BlockSpec((_BME, D), lambda i: (i, 0)),
      pl.BlockSpec((D, D), lambda i: (0, 0)),
      pl.BlockSpec((1, D), lambda i: (0, 0)),
  ]
  kwargs = {}
  if aliased:
    in_specs.append(pl.BlockSpec(memory_space=pl.ANY))
    kwargs["input_output_aliases"] = {3: 0}

  return pl.pallas_call(
      body,
      grid=(n_rows // _BME,),
      in_specs=in_specs,
      out_specs=pl.BlockSpec((_BME, D), lambda i: (boff + i, 0)),
      out_shape=jax.ShapeDtypeStruct((E, D), jnp.float32),
      **kwargs,
  )


_edge_mlps = []
_off = 0
for _i, _es in enumerate(ES):
  _edge_mlps.append(_make_edge_mlp(_es, _off, aliased=_i > 0))
  _off += _es


def kernel(x, edge_index, W1, b1, W2, b2, Wm1, bm1, Wm2, bm2):
  srcs = edge_index[0].reshape(NW, NCHUNK, CHUNK)
  dsts = edge_index[1].reshape(NW, NCHUNK, CHUNK)
  zeros = jnp.zeros((NPAD, D), jnp.float32)
  zerosd = jnp.zeros((NPAD, DEGW), jnp.float32)
  ones = jnp.ones((CHUNK, DEGW), jnp.float32)

  degp = _degree(dsts, ones, zerosd)
  agg1 = _seg_sum(x, srcs, dsts, zeros)
  h = _layer1(agg1, degp, W1, b1.reshape(1, D))
  agg2 = _seg_sum(h, srcs, dsts, zeros)
  a_nodes, b_nodes = _layer2(agg2, degp, W2, b2.reshape(1, D),
                             Wm1[:D], bm1.reshape(1, D), Wm1[D:])

  bm2r = bm2.reshape(1, D)
  out = None
  off = 0
  for i, (ng, es) in enumerate(zip(NGS, ES)):
    sl_src = edge_index[0, off:off + es].reshape(NW, ng * GRP, CCH)
    sl_dst = edge_index[1, off:off + es].reshape(NW, ng * GRP, CCH)
    c = _combine_by_ng[ng](a_nodes, b_nodes, sl_src, sl_dst)
    if i == 0:
      out = _edge_mlps[i](c, Wm2, bm2r)
    else:
      out = _edge_mlps[i](c, Wm2, bm2r, out)
    off += es
  return out


# speedup vs baseline: 1.1034x; 1.0058x over previous
import jax
import jax.numpy as jnp
from jax import lax
from jax.experimental import pallas as pl
from jax.experimental.pallas import tpu as pltpu
from jax.experimental.pallas import tpu_sc as plsc

N = 10000
E = 160000
D = 128
NC, NS = 2, 16
NW = NC * NS
EPW = E // NW
CHUNK = 125
NCHUNK = EPW // CHUNK
CCH = 40
NCCH = EPW // CCH
NPAD = 10240
RPT = NPAD // NS
DEGW = 128

_SC_MESH = plsc.VectorSubcoreMesh(
    core_axis_name="c", subcore_axis_name="s", num_cores=NC, num_subcores=NS)


def _seg_sum_body(x_hbm, srcs_hbm, dsts_hbm, zeros_hbm,
                  agg_hbm, idxs, idxd, rows0, rows1, acc, sem0, sem1):
  cid = lax.axis_index("c")
  sid = lax.axis_index("s")
  wid = cid * NS + sid
  pltpu.async_copy(zeros_hbm.at[pl.ds(sid * RPT, RPT)],
                   acc.at[pl.ds(sid * RPT, RPT)], sem0)
  pltpu.async_copy(srcs_hbm.at[wid], idxs, sem0)
  pltpu.async_copy(dsts_hbm.at[wid], idxd, sem0)
  pltpu.make_async_copy(zeros_hbm.at[pl.ds(sid * RPT, RPT)],
                        acc.at[pl.ds(sid * RPT, RPT)], sem0).wait()
  pltpu.make_async_copy(srcs_hbm.at[wid], idxs, sem0).wait()
  pltpu.make_async_copy(dsts_hbm.at[wid], idxd, sem0).wait()
  plsc.subcore_barrier()

  pltpu.async_copy(x_hbm.at[idxs.at[0]], rows0, sem0)
  pltpu.async_copy(x_hbm.at[idxs.at[1]], rows1, sem1)

  def body(p, carry):
    j0 = 2 * p
    pltpu.make_async_copy(x_hbm.at[idxs.at[j0]], rows0, sem0).wait()
    pltpu.sync_copy(rows0, acc.at[idxd.at[j0]], add=True)

    @pl.when(j0 + 2 < NCHUNK)
    def _():
      pltpu.async_copy(x_hbm.at[idxs.at[j0 + 2]], rows0, sem0)

    pltpu.make_async_copy(x_hbm.at[idxs.at[j0 + 1]], rows1, sem1).wait()
    pltpu.sync_copy(rows1, acc.at[idxd.at[j0 + 1]], add=True)

    @pl.when(j0 + 3 < NCHUNK)
    def _():
      pltpu.async_copy(x_hbm.at[idxs.at[j0 + 3]], rows1, sem1)

    return carry

  lax.fori_loop(0, NCHUNK // 2, body, 0)
  plsc.subcore_barrier()
  pltpu.sync_copy(acc.at[pl.ds(sid * RPT, RPT)],
                  agg_hbm.at[cid, pl.ds(sid * RPT, RPT)])


_seg_sum = pl.kernel(
    _seg_sum_body,
    out_type=jax.ShapeDtypeStruct((NC, NPAD, D), jnp.float32),
    mesh=_SC_MESH,
    scratch_types=[
        pltpu.VMEM((NCHUNK, CHUNK), jnp.int32),
        pltpu.VMEM((NCHUNK, CHUNK), jnp.int32),
        pltpu.VMEM((CHUNK, D), jnp.float32),
        pltpu.VMEM((CHUNK, D), jnp.float32),
        pltpu.VMEM_SHARED((NPAD, D), jnp.float32),
        pltpu.SemaphoreType.DMA,
        pltpu.SemaphoreType.DMA,
    ],
)


def _degree_body(dsts_hbm, ones_hbm, zerosd_hbm, deg_hbm,
                 idxd, ones_v, dacc, sem):
  cid = lax.axis_index("c")
  sid = lax.axis_index("s")
  wid = cid * NS + sid
  pltpu.async_copy(zerosd_hbm.at[pl.ds(sid * RPT, RPT)],
                   dacc.at[pl.ds(sid * RPT, RPT)], sem)
  pltpu.async_copy(dsts_hbm.at[wid], idxd, sem)
  pltpu.async_copy(ones_hbm, ones_v, sem)
  pltpu.make_async_copy(zerosd_hbm.at[pl.ds(sid * RPT, RPT)],
                        dacc.at[pl.ds(sid * RPT, RPT)], sem).wait()
  pltpu.make_async_copy(dsts_hbm.at[wid], idxd, sem).wait()
  pltpu.make_async_copy(ones_hbm, ones_v, sem).wait()
  plsc.subcore_barrier()

  def body(j, carry):
    pltpu.async_copy(ones_v, dacc.at[idxd.at[j]], sem, add=True)
    return carry

  lax.fori_loop(0, NCHUNK, body, 0)

  def drain(j, carry):
    pltpu.make_async_copy(ones_v, dacc.at[idxd.at[j]], sem).wait()
    return carry

  lax.fori_loop(0, NCHUNK, drain, 0)
  plsc.subcore_barrier()
  pltpu.sync_copy(dacc.at[pl.ds(sid * RPT, RPT)],
                  deg_hbm.at[cid, pl.ds(sid * RPT, RPT)])


_degree = pl.kernel(
    _degree_body,
    out_type=jax.ShapeDtypeStruct((NC, NPAD, DEGW), jnp.float32),
    mesh=_SC_MESH,
    scratch_types=[
        pltpu.VMEM((NCHUNK, CHUNK), jnp.int32),
        pltpu.VMEM((CHUNK, DEGW), jnp.float32),
        pltpu.VMEM_SHARED((NPAD, DEGW), jnp.float32),
        pltpu.SemaphoreType.DMA,
    ],
)


GRP = 5
NG = NCCH // GRP
GR = GRP * CCH
NGS = (10, 10, 5)


def _make_edge_combine(ng):
  nch = ng * GRP

  def body_fn(a_hbm, b_hbm, srcs_hbm, dsts_hbm, c_hbm,
              idxs, idxd, rows0, rows1, semA0, semA1, semB):
    cid = lax.axis_index("c")
    sid = lax.axis_index("s")
    wid = cid * NS + sid
    pltpu.async_copy(srcs_hbm.at[wid], idxs, semB)
    pltpu.async_copy(dsts_hbm.at[wid], idxd, semB)
    pltpu.make_async_copy(srcs_hbm.at[wid], idxs, semB).wait()
    pltpu.make_async_copy(dsts_hbm.at[wid], idxd, semB).wait()
    base = wid * (ng * GR)

    def issue_a(lg, buf, sem):
      for k in range(GRP):
        pltpu.async_copy(a_hbm.at[idxs.at[lg * GRP + k]],
                         buf.at[pl.ds(k * CCH, CCH)], sem)

    def process(lg, buf, sem):
      for k in range(GRP):
        pltpu.make_async_copy(a_hbm.at[idxs.at[lg * GRP + k]],
                              buf.at[pl.ds(k * CCH, CCH)], sem).wait()
      descs = [
          pltpu.async_copy(b_hbm.at[idxd.at[lg * GRP + k]],
                           buf.at[pl.ds(k * CCH, CCH)], semB, add=True)
          for k in range(GRP)
      ]
      for desc in descs:
        desc.wait()
      pltpu.sync_copy(buf, c_hbm.at[pl.ds(base + lg * GR, GR)])

    issue_a(0, rows0, semA0)
    if ng > 1:
      issue_a(1, rows1, semA1)

    def body(p, carry):
      lg0 = 2 * p
      process(lg0, rows0, semA0)

      @pl.when(lg0 + 2 < ng)
      def _():
        issue_a(lg0 + 2, rows0, semA0)

      process(lg0 + 1, rows1, semA1)

      @pl.when(lg0 + 3 < ng)
      def _():
        issue_a(lg0 + 3, rows1, semA1)

      return carry

    lax.fori_loop(0, ng // 2, body, 0)
    if ng % 2:
      process(ng - 1, rows0, semA0)

  return pl.kernel(
      body_fn,
      out_type=jax.ShapeDtypeStruct((NW * ng * GR, D), jnp.float32),
      mesh=_SC_MESH,
      scratch_types=[
          pltpu.VMEM((nch, CCH), jnp.int32),
          pltpu.VMEM((nch, CCH), jnp.int32),
          pltpu.VMEM((GR, D), jnp.float32),
          pltpu.VMEM((GR, D), jnp.float32),
          pltpu.SemaphoreType.DMA,
          pltpu.SemaphoreType.DMA,
          pltpu.SemaphoreType.DMA,
      ],
  )


_combine_by_ng = {ng: _make_edge_combine(ng) for ng in sorted(set(NGS))}
ES = tuple(NW * ng * GR for ng in NGS)
_BME = 1600


def _layer1_body(agg_ref, deg_ref, w_ref, b_ref, out_ref):
  deg = jnp.maximum(deg_ref[0, :, 0:1] + deg_ref[1, :, 0:1], 1.0)
  agg = (agg_ref[0] + agg_ref[1]) / deg
  h = jnp.dot(agg, w_ref[...], preferred_element_type=jnp.float32) + b_ref[...]
  out_ref[...] = jnp.maximum(h, 0.0)


def _layer1(agg, degp, w1, b1):
  return pl.pallas_call(
      _layer1_body,
      out_shape=jax.ShapeDtypeStruct((NPAD, D), jnp.float32),
  )(agg, degp, w1, b1)


def _layer2_body(agg_ref, deg_ref, w2_ref, b2_ref, wt_ref, bt_ref, wb_ref,
                 a_ref, b_ref):
  deg = jnp.maximum(deg_ref[0, :, 0:1] + deg_ref[1, :, 0:1], 1.0)
  agg = (agg_ref[0] + agg_ref[1]) / deg
  h2 = jnp.dot(agg, w2_ref[...], preferred_element_type=jnp.float32) + b2_ref[...]
  a_ref[...] = jnp.dot(h2, wt_ref[...],
                       preferred_element_type=jnp.float32) + bt_ref[...]
  b_ref[...] = jnp.dot(h2, wb_ref[...], preferred_element_type=jnp.float32)


def _layer2(agg, degp, w2, b2, wm1t, bm1, wm1b):
  return pl.pallas_call(
      _layer2_body,
      out_shape=(jax.ShapeDtypeStruct((NPAD, D), jnp.float32),
                 jax.ShapeDtypeStruct((NPAD, D), jnp.float32)),
  )(agg, degp, w2, b2, wm1t, bm1, wm1b)


def _make_edge_mlp(n_rows, row_off, aliased):
  boff = row_off // _BME

  def body(c_ref, w_ref, b_ref, *refs):
    o_ref = refs[-1]
    c = jnp.maximum(c_ref[...], 0.0)
    o_ref[...] = jnp.dot(c, w_ref[...],
                         preferred_element_type=jnp.float32) + b_ref[...]

  in_specs = [
      pl.BlockSpec((_BME, D), lambda i: (i, 0)),
      pl.BlockSpec((D, D), lambda i: (0, 0)),
      pl.BlockSpec((1, D), lambda i: (0, 0)),
  ]
  kwargs = {}
  if aliased:
    in_specs.append(pl.BlockSpec(memory_space=pl.ANY))
    kwargs["input_output_aliases"] = {3: 0}

  return pl.pallas_call(
      body,
      grid=(n_rows // _BME,),
      in_specs=in_specs,
      out_specs=pl.BlockSpec((_BME, D), lambda i: (boff + i, 0)),
      out_shape=jax.ShapeDtypeStruct((E, D), jnp.float32),
      **kwargs,
  )


_edge_mlps = []
_off = 0
for _i, _es in enumerate(ES):
  _edge_mlps.append(_make_edge_mlp(_es, _off, aliased=_i > 0))
  _off += _es


def kernel(x, edge_index, W1, b1, W2, b2, Wm1, bm1, Wm2, bm2):
  srcs = edge_index[0].reshape(NW, NCHUNK, CHUNK)
  dsts = edge_index[1].reshape(NW, NCHUNK, CHUNK)
  zeros = jnp.zeros((NPAD, D), jnp.float32)
  zerosd = jnp.zeros((NPAD, DEGW), jnp.float32)
  ones = jnp.ones((CHUNK, DEGW), jnp.float32)

  degp = _degree(dsts, ones, zerosd)
  agg1 = _seg_sum(x, srcs, dsts, zeros)
  h = _layer1(agg1, degp, W1, b1.reshape(1, D))
  agg2 = _seg_sum(h, srcs, dsts, zeros)
  a_nodes, b_nodes = _layer2(agg2, degp, W2, b2.reshape(1, D),
                             Wm1[:D], bm1.reshape(1, D), Wm1[D:])

  bm2r = bm2.reshape(1, D)
  out = None
  off = 0
  for i, (ng, es) in enumerate(zip(NGS, ES)):
    sl_src = edge_index[0, off:off + es].reshape(NW, ng * GRP, CCH)
    sl_dst = edge_index[1, off:off + es].reshape(NW, ng * GRP, CCH)
    c = _combine_by_ng[ng](a_nodes, b_nodes, sl_src, sl_dst)
    if i == 0:
      out = _edge_mlps[i](c, Wm2, bm2r)
    else:
      out = _edge_mlps[i](c, Wm2, bm2r, out)
    off += es
  return out
